# baseline (speedup 1.0000x reference)
import functools

import jax
import jax.numpy as jnp
from jax import lax
from jax.experimental import pallas as pl
from jax.experimental.pallas import tpu as pltpu
from jax.experimental.pallas import tpu_sc as plsc

N = 10000
E = 320000
D = 128
NC = 2
NS = 16
NW = NC * NS
CHUNK = 128
CPT = 80
TOTC = NW * CPT
EPAD = TOTC * CHUNK
NPAD = 10240
RPT = NPAD // NS
RB = 1024
GRID = NPAD // RB
SLAB = 16
CPT_C0 = 160
CPT_C1 = 160 - CPT_C0

_mesh = plsc.VectorSubcoreMesh(core_axis_name="c", subcore_axis_name="s")


@functools.partial(
    pl.kernel,
    mesh=_mesh,
    out_type=jax.ShapeDtypeStruct((NC, NPAD, D), jnp.float32),
    scratch_types=[
        pltpu.VMEM((CPT, CHUNK), jnp.int32),
        pltpu.VMEM((CHUNK, D), jnp.float32),
        pltpu.VMEM((CHUNK, D), jnp.float32),
        pltpu.VMEM_SHARED((NPAD, D), jnp.float32),
    ],
)
def _deg_kernel(dst_hbm, deg_hbm, idx_v, ones_v, zero_v, acc):
    c = lax.axis_index("c")
    s = lax.axis_index("s")
    wid = c * NS + s

    def fill_ones(i, _):
        for k in range(D // 16):
            ones_v[i, pl.ds(k * 16, 16)] = jnp.full((16,), 1.0, jnp.float32)
        return 0

    lax.fori_loop(0, CHUNK, fill_ones, 0)

    def fill_zero(i, _):
        for k in range(D // 16):
            zero_v[i, pl.ds(k * 16, 16)] = jnp.zeros((16,), jnp.float32)
        return 0

    lax.fori_loop(0, CHUNK, fill_zero, 0)

    def zero_acc(t, _):
        pltpu.sync_copy(zero_v, acc.at[pl.ds(s * RPT + t * CHUNK, CHUNK)])
        return 0

    lax.fori_loop(0, RPT // CHUNK, zero_acc, 0)

    pltpu.sync_copy(dst_hbm.at[pl.ds(wid * CPT, CPT)], idx_v)
    plsc.subcore_barrier()

    def body(j, _):
        pltpu.sync_copy(ones_v, acc.at[idx_v.at[j]], add=True)
        return 0

    lax.fori_loop(0, CPT, body, 0)

    plsc.subcore_barrier()

    def copy_out(t, _):
        pltpu.sync_copy(acc.at[pl.ds(s * RPT + t * CHUNK, CHUNK)],
                        deg_hbm.at[c, pl.ds(s * RPT + t * CHUNK, CHUNK)])
        return 0

    lax.fori_loop(0, RPT // CHUNK, copy_out, 0)


NBUF = 2


@functools.partial(
    pl.kernel,
    mesh=_mesh,
    out_type=jax.ShapeDtypeStruct((NC, NPAD, D), jnp.float32),
    scratch_types=[
        pltpu.VMEM((SLAB, CHUNK), jnp.int32),
        pltpu.VMEM((SLAB, CHUNK), jnp.int32),
        pltpu.VMEM_SHARED((NPAD, D), jnp.float32),
    ]
    + [pltpu.VMEM((CHUNK, D), jnp.float32)] * NBUF
    + [pltpu.SemaphoreType.DMA] * NBUF,
)
def _agg_kernel(h_hbm, src_hbm, dst_hbm, out_hbm, sidx, didx, acc, *bufsem):
    bufs = bufsem[:NBUF]
    gsem = bufsem[NBUF:]
    c = lax.axis_index("c")
    s = lax.axis_index("s")
    base = jnp.where(c == 0, s * CPT_C0, NS * CPT_C0 + s * CPT_C1)
    nslab = jnp.where(c == 0, CPT_C0 // SLAB, CPT_C1 // SLAB)

    def zero_buf(r, _):
        for k in range(D // 16):
            bufs[0][r, pl.ds(k * 16, 16)] = jnp.zeros((16,), jnp.float32)
        return 0

    lax.fori_loop(0, CHUNK, zero_buf, 0)

    def zero_acc(t, _):
        pltpu.sync_copy(bufs[0], acc.at[pl.ds(s * RPT + t * CHUNK, CHUNK)])
        return 0

    lax.fori_loop(0, RPT // CHUNK, zero_acc, 0)
    plsc.subcore_barrier()

    def slab_body(p, _):
        off = pl.multiple_of(base + p * SLAB, SLAB)
        pltpu.sync_copy(src_hbm.at[pl.ds(off, SLAB)], sidx)
        pltpu.sync_copy(dst_hbm.at[pl.ds(off, SLAB)], didx)

        for b in range(NBUF):
            pltpu.async_copy(h_hbm.at[sidx.at[b]], bufs[b], gsem[b])

        def body(g, _):
            for b in range(NBUF):
                j = g * NBUF + b
                pltpu.make_async_copy(h_hbm.at[sidx.at[j]], bufs[b],
                                      gsem[b]).wait()
                pltpu.sync_copy(bufs[b], acc.at[didx.at[j]], add=True)
                pltpu.async_copy(h_hbm.at[sidx.at[j + NBUF]], bufs[b], gsem[b])
            return 0

        lax.fori_loop(0, SLAB // NBUF - 1, body, 0)

        for b in range(NBUF):
            j = SLAB - NBUF + b
            pltpu.make_async_copy(h_hbm.at[sidx.at[j]], bufs[b], gsem[b]).wait()
            pltpu.sync_copy(bufs[b], acc.at[didx.at[j]], add=True)
        return 0

    lax.fori_loop(0, nslab, slab_body, 0)

    plsc.subcore_barrier()

    def copy_out(t, _):
        pltpu.sync_copy(acc.at[pl.ds(s * RPT + t * CHUNK, CHUNK)],
                        out_hbm.at[c, pl.ds(s * RPT + t * CHUNK, CHUNK)])
        return 0

    lax.fori_loop(0, RPT // CHUNK, copy_out, 0)


def _k1_body(deg_ref, x_ref, w_ref, h_ref, dis_ref):
    i = pl.program_id(0)
    d2 = deg_ref[0, :, 0:1] + deg_ref[1, :, 0:1] + 1.0
    r2 = lax.broadcasted_iota(jnp.int32, (RB, 1), 0) + i * RB
    dis2 = jnp.where(r2 < N, lax.rsqrt(d2), 0.0)
    h = jnp.dot(x_ref[...], w_ref[...], preferred_element_type=jnp.float32,
                precision=lax.Precision.HIGHEST)
    h_ref[...] = h * dis2
    dis_ref[...] = dis2


_k1 = pl.pallas_call(
    _k1_body,
    grid=(GRID,),
    in_specs=[
        pl.BlockSpec((NC, RB, D), lambda i: (0, i, 0)),
        pl.BlockSpec((RB, D), lambda i: (i, 0)),
        pl.BlockSpec((D, D), lambda i: (0, 0)),
    ],
    out_specs=[
        pl.BlockSpec((RB, D), lambda i: (i, 0)),
        pl.BlockSpec((RB, 1), lambda i: (i, 0)),
    ],
    out_shape=[
        jax.ShapeDtypeStruct((NPAD, D), jnp.float32),
        jax.ShapeDtypeStruct((NPAD, 1), jnp.float32),
    ],
)


def _k2_body(part_ref, h1_ref, dis_ref, b1_ref, w2_ref, h2_ref):
    ssum = part_ref[0] + part_ref[1] + h1_ref[...]
    o1 = ssum * dis_ref[...] + b1_ref[...]
    a = jnp.maximum(o1, 0.0)
    h2_ref[...] = jnp.dot(a, w2_ref[...], preferred_element_type=jnp.float32,
                          precision=lax.Precision.HIGHEST) * dis_ref[...]


_k2 = pl.pallas_call(
    _k2_body,
    grid=(GRID,),
    in_specs=[
        pl.BlockSpec((NC, RB, D), lambda i: (0, i, 0)),
        pl.BlockSpec((RB, D), lambda i: (i, 0)),
        pl.BlockSpec((RB, 1), lambda i: (i, 0)),
        pl.BlockSpec((1, D), lambda i: (0, 0)),
        pl.BlockSpec((D, D), lambda i: (0, 0)),
    ],
    out_specs=pl.BlockSpec((RB, D), lambda i: (i, 0)),
    out_shape=jax.ShapeDtypeStruct((NPAD, D), jnp.float32),
)


def _k3_body(part_ref, h2_ref, dis_ref, b2_ref, out_ref):
    ssum = part_ref[0] + part_ref[1] + h2_ref[...]
    o = ssum * dis_ref[...] + b2_ref[...]
    m = jnp.max(o, axis=1, keepdims=True)
    ex = jnp.exp(o - m)
    lse = jnp.log(jnp.sum(ex, axis=1, keepdims=True))
    out_ref[...] = (o - m) - lse


_k3 = pl.pallas_call(
    _k3_body,
    grid=(GRID,),
    in_specs=[
        pl.BlockSpec((NC, RB, D), lambda i: (0, i, 0)),
        pl.BlockSpec((RB, D), lambda i: (i, 0)),
        pl.BlockSpec((RB, 1), lambda i: (i, 0)),
        pl.BlockSpec((1, D), lambda i: (0, 0)),
    ],
    out_specs=pl.BlockSpec((RB, D), lambda i: (i, 0)),
    out_shape=jax.ShapeDtypeStruct((NPAD, D), jnp.float32),
)


def kernel(x, edge_index, W1, b1, W2, b2):
    ei = edge_index.astype(jnp.int32)
    pad = jnp.full((EPAD - E,), N, jnp.int32)
    src2 = jnp.concatenate([ei[0], pad]).reshape(TOTC, CHUNK)
    dst2 = jnp.concatenate([ei[1], pad]).reshape(TOTC, CHUNK)
    xp = jnp.pad(x, ((0, NPAD - N), (0, 0)))
    b1r = b1.reshape(1, D)
    b2r = b2.reshape(1, D)

    degp = _deg_kernel(dst2)
    h1p, dis = _k1(degp, xp, W1)
    part1 = _agg_kernel(h1p, src2, dst2)
    h2p = _k2(part1, h1p, dis, b1r, W2)
    part2 = _agg_kernel(h2p, src2, dst2)
    outp = _k3(part2, h2p, dis, b2r)
    return outp[:N]


# speedup vs baseline: 1.0758x; 1.0758x over previous
import functools

import jax
import jax.numpy as jnp
from jax import lax
from jax.experimental import pallas as pl
from jax.experimental.pallas import tpu as pltpu
from jax.experimental.pallas import tpu_sc as plsc

N = 10000
E = 320000
D = 128
NC = 2
NS = 16
NW = NC * NS
CHUNK = 128
CPT = 80
TOTC = NW * CPT
EPAD = TOTC * CHUNK
NPAD = 10240
RPT = NPAD // NS
RB = 1024
GRID = NPAD // RB
SLAB = 16
CPT_C0 = 80
CPT_C1 = 160 - CPT_C0

_mesh = plsc.VectorSubcoreMesh(core_axis_name="c", subcore_axis_name="s")


@functools.partial(
    pl.kernel,
    mesh=_mesh,
    out_type=jax.ShapeDtypeStruct((NC, NPAD, D), jnp.float32),
    scratch_types=[
        pltpu.VMEM((CPT, CHUNK), jnp.int32),
        pltpu.VMEM((CHUNK, D), jnp.float32),
        pltpu.VMEM((CHUNK, D), jnp.float32),
        pltpu.VMEM_SHARED((NPAD, D), jnp.float32),
    ],
)
def _deg_kernel(dst_hbm, deg_hbm, idx_v, ones_v, zero_v, acc):
    c = lax.axis_index("c")
    s = lax.axis_index("s")
    wid = c * NS + s

    def fill_ones(i, _):
        for k in range(D // 16):
            ones_v[i, pl.ds(k * 16, 16)] = jnp.full((16,), 1.0, jnp.float32)
        return 0

    lax.fori_loop(0, CHUNK, fill_ones, 0)

    def fill_zero(i, _):
        for k in range(D // 16):
            zero_v[i, pl.ds(k * 16, 16)] = jnp.zeros((16,), jnp.float32)
        return 0

    lax.fori_loop(0, CHUNK, fill_zero, 0)

    def zero_acc(t, _):
        pltpu.sync_copy(zero_v, acc.at[pl.ds(s * RPT + t * CHUNK, CHUNK)])
        return 0

    lax.fori_loop(0, RPT // CHUNK, zero_acc, 0)

    pltpu.sync_copy(dst_hbm.at[pl.ds(wid * CPT, CPT)], idx_v)
    plsc.subcore_barrier()

    def body(j, _):
        pltpu.sync_copy(ones_v, acc.at[idx_v.at[j]], add=True)
        return 0

    lax.fori_loop(0, CPT, body, 0)

    plsc.subcore_barrier()

    def copy_out(t, _):
        pltpu.sync_copy(acc.at[pl.ds(s * RPT + t * CHUNK, CHUNK)],
                        deg_hbm.at[c, pl.ds(s * RPT + t * CHUNK, CHUNK)])
        return 0

    lax.fori_loop(0, RPT // CHUNK, copy_out, 0)


NBUF = 2


@functools.partial(
    pl.kernel,
    mesh=_mesh,
    out_type=jax.ShapeDtypeStruct((NC, NPAD, D), jnp.float32),
    scratch_types=[
        pltpu.VMEM((SLAB, CHUNK), jnp.int32),
        pltpu.VMEM((SLAB, CHUNK), jnp.int32),
        pltpu.VMEM_SHARED((NPAD, D), jnp.float32),
    ]
    + [pltpu.VMEM((CHUNK, D), jnp.float32)] * NBUF
    + [pltpu.SemaphoreType.DMA] * NBUF,
)
def _agg_kernel(h_hbm, src_hbm, dst_hbm, out_hbm, sidx, didx, acc, *bufsem):
    bufs = bufsem[:NBUF]
    gsem = bufsem[NBUF:]
    c = lax.axis_index("c")
    s = lax.axis_index("s")
    base = jnp.where(c == 0, s * CPT_C0, NS * CPT_C0 + s * CPT_C1)
    nslab = jnp.where(c == 0, CPT_C0 // SLAB, CPT_C1 // SLAB)

    def zero_buf(r, _):
        for k in range(D // 16):
            bufs[0][r, pl.ds(k * 16, 16)] = jnp.zeros((16,), jnp.float32)
        return 0

    lax.fori_loop(0, CHUNK, zero_buf, 0)

    def zero_acc(t, _):
        pltpu.sync_copy(bufs[0], acc.at[pl.ds(s * RPT + t * CHUNK, CHUNK)])
        return 0

    lax.fori_loop(0, RPT // CHUNK, zero_acc, 0)
    plsc.subcore_barrier()

    def slab_body(p, _):
        off = pl.multiple_of(base + p * SLAB, SLAB)
        pltpu.sync_copy(src_hbm.at[pl.ds(off, SLAB)], sidx)
        pltpu.sync_copy(dst_hbm.at[pl.ds(off, SLAB)], didx)

        for b in range(NBUF):
            pltpu.async_copy(h_hbm.at[sidx.at[b]], bufs[b], gsem[b])

        def body(g, _):
            for b in range(NBUF):
                j = g * NBUF + b
                pltpu.make_async_copy(h_hbm.at[sidx.at[j]], bufs[b],
                                      gsem[b]).wait()
                pltpu.sync_copy(bufs[b], acc.at[didx.at[j]], add=True)
                pltpu.async_copy(h_hbm.at[sidx.at[j + NBUF]], bufs[b], gsem[b])
            return 0

        lax.fori_loop(0, SLAB // NBUF - 1, body, 0)

        for b in range(NBUF):
            j = SLAB - NBUF + b
            pltpu.make_async_copy(h_hbm.at[sidx.at[j]], bufs[b], gsem[b]).wait()
            pltpu.sync_copy(bufs[b], acc.at[didx.at[j]], add=True)
        return 0

    lax.fori_loop(0, nslab, slab_body, 0)

    plsc.subcore_barrier()

    def copy_out(t, _):
        pltpu.sync_copy(acc.at[pl.ds(s * RPT + t * CHUNK, CHUNK)],
                        out_hbm.at[c, pl.ds(s * RPT + t * CHUNK, CHUNK)])
        return 0

    lax.fori_loop(0, RPT // CHUNK, copy_out, 0)


def _k0_body(x_ref, w_ref, h_ref):
    h_ref[...] = jnp.dot(x_ref[...], w_ref[...],
                         preferred_element_type=jnp.float32,
                         precision=lax.Precision.HIGHEST)


_k0 = pl.pallas_call(
    _k0_body,
    grid=(GRID,),
    in_specs=[
        pl.BlockSpec((RB, D), lambda i: (i, 0)),
        pl.BlockSpec((D, D), lambda i: (0, 0)),
    ],
    out_specs=pl.BlockSpec((RB, D), lambda i: (i, 0)),
    out_shape=jax.ShapeDtypeStruct((NPAD, D), jnp.float32),
)


def _k1_body(deg_ref, h_ref, hs_ref, dis_ref):
    i = pl.program_id(0)
    d2 = deg_ref[0, :, 0:1] + deg_ref[1, :, 0:1] + 1.0
    r2 = lax.broadcasted_iota(jnp.int32, (RB, 1), 0) + i * RB
    dis2 = jnp.where(r2 < N, lax.rsqrt(d2), 0.0)
    hs_ref[...] = h_ref[...] * dis2
    dis_ref[...] = dis2


_k1 = pl.pallas_call(
    _k1_body,
    grid=(GRID,),
    in_specs=[
        pl.BlockSpec((NC, RB, D), lambda i: (0, i, 0)),
        pl.BlockSpec((RB, D), lambda i: (i, 0)),
    ],
    out_specs=[
        pl.BlockSpec((RB, D), lambda i: (i, 0)),
        pl.BlockSpec((RB, 1), lambda i: (i, 0)),
    ],
    out_shape=[
        jax.ShapeDtypeStruct((NPAD, D), jnp.float32),
        jax.ShapeDtypeStruct((NPAD, 1), jnp.float32),
    ],
)


def _k2_body(part_ref, h1_ref, dis_ref, b1_ref, w2_ref, h2_ref):
    ssum = part_ref[0] + part_ref[1] + h1_ref[...]
    o1 = ssum * dis_ref[...] + b1_ref[...]
    a = jnp.maximum(o1, 0.0)
    h2_ref[...] = jnp.dot(a, w2_ref[...], preferred_element_type=jnp.float32,
                          precision=lax.Precision.HIGHEST) * dis_ref[...]


_k2 = pl.pallas_call(
    _k2_body,
    grid=(GRID,),
    in_specs=[
        pl.BlockSpec((NC, RB, D), lambda i: (0, i, 0)),
        pl.BlockSpec((RB, D), lambda i: (i, 0)),
        pl.BlockSpec((RB, 1), lambda i: (i, 0)),
        pl.BlockSpec((1, D), lambda i: (0, 0)),
        pl.BlockSpec((D, D), lambda i: (0, 0)),
    ],
    out_specs=pl.BlockSpec((RB, D), lambda i: (i, 0)),
    out_shape=jax.ShapeDtypeStruct((NPAD, D), jnp.float32),
)


def _k3_body(part_ref, h2_ref, dis_ref, b2_ref, out_ref):
    ssum = part_ref[0] + part_ref[1] + h2_ref[...]
    o = ssum * dis_ref[...] + b2_ref[...]
    m = jnp.max(o, axis=1, keepdims=True)
    ex = jnp.exp(o - m)
    lse = jnp.log(jnp.sum(ex, axis=1, keepdims=True))
    out_ref[...] = (o - m) - lse


_k3 = pl.pallas_call(
    _k3_body,
    grid=(GRID,),
    in_specs=[
        pl.BlockSpec((NC, RB, D), lambda i: (0, i, 0)),
        pl.BlockSpec((RB, D), lambda i: (i, 0)),
        pl.BlockSpec((RB, 1), lambda i: (i, 0)),
        pl.BlockSpec((1, D), lambda i: (0, 0)),
    ],
    out_specs=pl.BlockSpec((RB, D), lambda i: (i, 0)),
    out_shape=jax.ShapeDtypeStruct((NPAD, D), jnp.float32),
)


def kernel(x, edge_index, W1, b1, W2, b2):
    ei = edge_index.astype(jnp.int32)
    pad = jnp.full((EPAD - E,), N, jnp.int32)
    src2 = jnp.concatenate([ei[0], pad]).reshape(TOTC, CHUNK)
    dst2 = jnp.concatenate([ei[1], pad]).reshape(TOTC, CHUNK)
    xp = jnp.pad(x, ((0, NPAD - N), (0, 0)))
    b1r = b1.reshape(1, D)
    b2r = b2.reshape(1, D)

    h1raw = _k0(xp, W1)
    degp = _deg_kernel(dst2)
    h1p, dis = _k1(degp, h1raw)
    part1 = _agg_kernel(h1p, src2, dst2)
    h2p = _k2(part1, h1p, dis, b1r, W2)
    part2 = _agg_kernel(h2p, src2, dst2)
    outp = _k3(part2, h2p, dis, b2r)
    return outp[:N]


# speedup vs baseline: 1.3787x; 1.2816x over previous
import functools

import jax
import jax.numpy as jnp
from jax import lax
from jax.experimental import pallas as pl
from jax.experimental.pallas import tpu as pltpu
from jax.experimental.pallas import tpu_sc as plsc

N = 10000
E = 320000
D = 128
NC = 2
NS = 16
NW = NC * NS
CHUNK = 128
CPT = 80
TOTC = NW * CPT
EPAD = TOTC * CHUNK
NPAD = 10240
RPT = NPAD // NS
RB = 1024
GRID = NPAD // RB
SLAB = 32
CPT_C0 = 80
CPT_C1 = 160 - CPT_C0

_mesh = plsc.VectorSubcoreMesh(core_axis_name="c", subcore_axis_name="s")


@functools.partial(
    pl.kernel,
    mesh=_mesh,
    out_type=jax.ShapeDtypeStruct((NC, NPAD, D), jnp.float32),
    scratch_types=[
        pltpu.VMEM((CPT, CHUNK), jnp.int32),
        pltpu.VMEM((CHUNK, D), jnp.float32),
        pltpu.VMEM((CHUNK, D), jnp.float32),
        pltpu.VMEM_SHARED((NPAD, D), jnp.float32),
    ],
)
def _deg_kernel(dst_hbm, deg_hbm, idx_v, ones_v, zero_v, acc):
    c = lax.axis_index("c")
    s = lax.axis_index("s")
    wid = c * NS + s

    def fill_ones(i, _):
        for k in range(D // 16):
            ones_v[i, pl.ds(k * 16, 16)] = jnp.full((16,), 1.0, jnp.float32)
        return 0

    lax.fori_loop(0, CHUNK, fill_ones, 0)

    def fill_zero(i, _):
        for k in range(D // 16):
            zero_v[i, pl.ds(k * 16, 16)] = jnp.zeros((16,), jnp.float32)
        return 0

    lax.fori_loop(0, CHUNK, fill_zero, 0)

    def zero_acc(t, _):
        pltpu.sync_copy(zero_v, acc.at[pl.ds(s * RPT + t * CHUNK, CHUNK)])
        return 0

    lax.fori_loop(0, RPT // CHUNK, zero_acc, 0)

    pltpu.sync_copy(dst_hbm.at[pl.ds(wid * CPT, CPT)], idx_v)
    plsc.subcore_barrier()

    def body(j, _):
        pltpu.sync_copy(ones_v, acc.at[idx_v.at[j]], add=True)
        return 0

    lax.fori_loop(0, CPT, body, 0)

    plsc.subcore_barrier()

    def copy_out(t, _):
        pltpu.sync_copy(acc.at[pl.ds(s * RPT + t * CHUNK, CHUNK)],
                        deg_hbm.at[c, pl.ds(s * RPT + t * CHUNK, CHUNK)])
        return 0

    lax.fori_loop(0, RPT // CHUNK, copy_out, 0)


NBUF = 2


@functools.partial(
    pl.kernel,
    mesh=_mesh,
    out_type=jax.ShapeDtypeStruct((NC, NPAD, D), jnp.float32),
    scratch_types=[
        pltpu.VMEM((SLAB, CHUNK), jnp.int32),
        pltpu.VMEM((SLAB, CHUNK), jnp.int32),
        pltpu.VMEM_SHARED((NPAD, D), jnp.float32),
    ]
    + [pltpu.VMEM((CHUNK, D), jnp.float32)] * NBUF
    + [pltpu.SemaphoreType.DMA] * NBUF,
)
def _agg_kernel(h_hbm, src_hbm, dst_hbm, out_hbm, sidx, didx, acc, *bufsem):
    bufs = bufsem[:NBUF]
    gsem = bufsem[NBUF:]
    c = lax.axis_index("c")
    s = lax.axis_index("s")
    base = jnp.where(c == 0, s * CPT_C0, NS * CPT_C0 + s * CPT_C1)
    nslab = jnp.where(c == 0, CPT_C0 // SLAB, CPT_C1 // SLAB)

    def zero_buf(r, _):
        for k in range(D // 16):
            bufs[0][r, pl.ds(k * 16, 16)] = jnp.zeros((16,), jnp.float32)
        return 0

    lax.fori_loop(0, CHUNK, zero_buf, 0)

    def zero_acc(t, _):
        pltpu.sync_copy(bufs[0], acc.at[pl.ds(s * RPT + t * CHUNK, CHUNK)])
        return 0

    lax.fori_loop(0, RPT // CHUNK, zero_acc, 0)
    plsc.subcore_barrier()

    def slab_body(p, _):
        off = pl.multiple_of(base + p * SLAB, SLAB)
        pltpu.sync_copy(src_hbm.at[pl.ds(off, SLAB)], sidx)
        pltpu.sync_copy(dst_hbm.at[pl.ds(off, SLAB)], didx)

        for b in range(NBUF):
            pltpu.async_copy(h_hbm.at[sidx.at[b]], bufs[b], gsem[b])

        def body(g, _):
            for b in range(NBUF):
                j = g * NBUF + b
                pltpu.make_async_copy(h_hbm.at[sidx.at[j]], bufs[b],
                                      gsem[b]).wait()
                pltpu.sync_copy(bufs[b], acc.at[didx.at[j]], add=True)
                pltpu.async_copy(h_hbm.at[sidx.at[j + NBUF]], bufs[b], gsem[b])
            return 0

        lax.fori_loop(0, SLAB // NBUF - 1, body, 0)

        for b in range(NBUF):
            j = SLAB - NBUF + b
            pltpu.make_async_copy(h_hbm.at[sidx.at[j]], bufs[b], gsem[b]).wait()
            pltpu.sync_copy(bufs[b], acc.at[didx.at[j]], add=True)
        return 0

    lax.fori_loop(0, nslab, slab_body, 0)

    plsc.subcore_barrier()

    def copy_out(t, _):
        pltpu.sync_copy(acc.at[pl.ds(s * RPT + t * CHUNK, CHUNK)],
                        out_hbm.at[c, pl.ds(s * RPT + t * CHUNK, CHUNK)])
        return 0

    lax.fori_loop(0, RPT // CHUNK, copy_out, 0)


def _k1_body(deg_ref, x_ref, w_ref, h_ref, dis_ref):
    i = pl.program_id(0)
    d2 = deg_ref[0, :, 0:1] + deg_ref[1, :, 0:1] + 1.0
    r2 = lax.broadcasted_iota(jnp.int32, (RB, 1), 0) + i * RB
    dis2 = jnp.where(r2 < N, lax.rsqrt(d2), 0.0)
    h = jnp.dot(x_ref[...], w_ref[...], preferred_element_type=jnp.float32,
                precision=lax.Precision.HIGHEST)
    h_ref[...] = h * dis2
    dis_ref[...] = dis2


_k1 = pl.pallas_call(
    _k1_body,
    grid=(GRID,),
    in_specs=[
        pl.BlockSpec((NC, RB, D), lambda i: (0, i, 0)),
        pl.BlockSpec((RB, D), lambda i: (i, 0)),
        pl.BlockSpec((D, D), lambda i: (0, 0)),
    ],
    out_specs=[
        pl.BlockSpec((RB, D), lambda i: (i, 0)),
        pl.BlockSpec((RB, 1), lambda i: (i, 0)),
    ],
    out_shape=[
        jax.ShapeDtypeStruct((NPAD, D), jnp.float32),
        jax.ShapeDtypeStruct((NPAD, 1), jnp.float32),
    ],
)


def _k2_body(part_ref, h1_ref, dis_ref, b1_ref, w2_ref, h2_ref):
    ssum = part_ref[0] + part_ref[1] + h1_ref[...]
    o1 = ssum * dis_ref[...] + b1_ref[...]
    a = jnp.maximum(o1, 0.0)
    h2_ref[...] = jnp.dot(a, w2_ref[...], preferred_element_type=jnp.float32,
                          precision=lax.Precision.HIGHEST) * dis_ref[...]


_k2 = pl.pallas_call(
    _k2_body,
    grid=(GRID,),
    in_specs=[
        pl.BlockSpec((NC, RB, D), lambda i: (0, i, 0)),
        pl.BlockSpec((RB, D), lambda i: (i, 0)),
        pl.BlockSpec((RB, 1), lambda i: (i, 0)),
        pl.BlockSpec((1, D), lambda i: (0, 0)),
        pl.BlockSpec((D, D), lambda i: (0, 0)),
    ],
    out_specs=pl.BlockSpec((RB, D), lambda i: (i, 0)),
    out_shape=jax.ShapeDtypeStruct((NPAD, D), jnp.float32),
)


def _k3_body(part_ref, h2_ref, dis_ref, b2_ref, out_ref):
    ssum = part_ref[0] + part_ref[1] + h2_ref[...]
    o = ssum * dis_ref[...] + b2_ref[...]
    m = jnp.max(o, axis=1, keepdims=True)
    ex = jnp.exp(o - m)
    lse = jnp.log(jnp.sum(ex, axis=1, keepdims=True))
    out_ref[...] = (o - m) - lse


_k3 = pl.pallas_call(
    _k3_body,
    grid=(GRID,),
    in_specs=[
        pl.BlockSpec((NC, RB, D), lambda i: (0, i, 0)),
        pl.BlockSpec((RB, D), lambda i: (i, 0)),
        pl.BlockSpec((RB, 1), lambda i: (i, 0)),
        pl.BlockSpec((1, D), lambda i: (0, 0)),
    ],
    out_specs=pl.BlockSpec((RB, D), lambda i: (i, 0)),
    out_shape=jax.ShapeDtypeStruct((NPAD, D), jnp.float32),
)


def kernel(x, edge_index, W1, b1, W2, b2):
    ei = edge_index.astype(jnp.int32)
    pad = jnp.full((EPAD - E,), N, jnp.int32)
    src2 = jnp.concatenate([ei[0], pad]).reshape(TOTC, CHUNK)
    dst2 = jnp.concatenate([ei[1], pad]).reshape(TOTC, CHUNK)
    xp = jnp.pad(x, ((0, NPAD - N), (0, 0)))
    b1r = b1.reshape(1, D)
    b2r = b2.reshape(1, D)

    degp = _deg_kernel(dst2)
    h1p, dis = _k1(degp, xp, W1)
    part1 = _agg_kernel(h1p, src2, dst2)
    h2p = _k2(part1, h1p, dis, b1r, W2)
    part2 = _agg_kernel(h2p, src2, dst2)
    outp = _k3(part2, h2p, dis, b2r)
    return outp[:N]
